# baseline (speedup 1.0000x reference)
import functools

import jax
import jax.numpy as jnp
from jax import lax
from jax.experimental import pallas as pl
from jax.experimental.pallas import tpu as pltpu
from jax.experimental.pallas import tpu_sc as plsc

NC = 2
NS = 16
NW = NC * NS
LANES = 16
NBUF = 2


def _sc_aggregate(nodes, adj, features):
  B = nodes.shape[0]
  DEG = adj.shape[1]
  D = features.shape[1]
  assert B % NW == 0
  b_per_w = B // NW
  scale = 1.0 / DEG
  n_chunks = D // LANES

  mesh = plsc.VectorSubcoreMesh(core_axis_name="c", subcore_axis_name="s",
                                num_cores=NC, num_subcores=NS)

  @functools.partial(
      pl.kernel,
      mesh=mesh,
      compiler_params=pltpu.CompilerParams(use_tc_tiling_on_sc=False),
      out_type=jax.ShapeDtypeStruct((B, D), jnp.float32),
      scratch_types=[
          pltpu.VMEM((b_per_w,), jnp.int32),
          pltpu.VMEM((b_per_w, DEG), jnp.int32),
          pltpu.VMEM((NBUF, DEG, D), jnp.float32),
          pltpu.VMEM((b_per_w, D), jnp.float32),
          pltpu.SemaphoreType.DMA,
          pltpu.SemaphoreType.DMA((NBUF,)),
      ],
  )
  def agg_kernel(nodes_hbm, adj_hbm, feat_hbm, out_hbm,
                 nodes_v, adjrows_v, bufs_v, agg_v, sem0, gsems):
    wid = lax.axis_index("s") * NC + lax.axis_index("c")
    base = wid * b_per_w

    pltpu.sync_copy(nodes_hbm.at[pl.ds(base, b_per_w)], nodes_v)
    pltpu.async_copy(adj_hbm.at[nodes_v], adjrows_v, sem0).wait()

    def start_gather(item, k):
      pltpu.async_copy(feat_hbm.at[adjrows_v.at[item]], bufs_v.at[k],
                       gsems.at[k])

    for k in range(NBUF):
      start_gather(k, k)

    def ring_body(g, _):
      for k in range(NBUF):
        item = g * NBUF + k
        pltpu.make_async_copy(feat_hbm.at[adjrows_v.at[item]], bufs_v.at[k],
                              gsems.at[k]).wait()
        acc = [bufs_v[k, 0, pl.ds(c * LANES, LANES)] for c in range(n_chunks)]
        for r in range(1, DEG):
          for c in range(n_chunks):
            acc[c] = acc[c] + bufs_v[k, r, pl.ds(c * LANES, LANES)]
        for c in range(n_chunks):
          agg_v[item, pl.ds(c * LANES, LANES)] = acc[c] * scale

        @pl.when(item + NBUF < b_per_w)
        def _():
          start_gather(item + NBUF, k)
      return 0

    lax.fori_loop(0, b_per_w // NBUF, ring_body, 0)
    pltpu.sync_copy(agg_v, out_hbm.at[pl.ds(base, b_per_w)])

  return agg_kernel(nodes, adj, features)


def _tc_tail(agg, W):
  B, D = agg.shape
  BLK = 2048
  grid = B // BLK

  def body(a_ref, w_ref, o_ref):
    h = jnp.dot(a_ref[...], w_ref[...], preferred_element_type=jnp.float32)
    h = jnp.maximum(h, 0.0)
    norm = jnp.sqrt(jnp.sum(h * h, axis=1, keepdims=True))
    o_ref[...] = h / jnp.maximum(norm, 1e-12)

  return pl.pallas_call(
      body,
      grid=(grid,),
      in_specs=[
          pl.BlockSpec((BLK, D), lambda i: (i, 0)),
          pl.BlockSpec((D, D), lambda i: (0, 0)),
      ],
      out_specs=pl.BlockSpec((BLK, D), lambda i: (i, 0)),
      out_shape=jax.ShapeDtypeStruct((B, D), jnp.float32),
  )(agg, W)


@jax.jit
def kernel(nodes, features, adj, W):
  nodes = nodes.astype(jnp.int32)
  agg = _sc_aggregate(nodes, adj, features)
  return _tc_tail(agg, W)


# speedup vs baseline: 1.8201x; 1.8201x over previous
import functools

import jax
import jax.numpy as jnp
from jax import lax
from jax.experimental import pallas as pl
from jax.experimental.pallas import tpu as pltpu
from jax.experimental.pallas import tpu_sc as plsc

NC = 2
NS = 16
NW = NC * NS
LANES = 16
NQ = 4


def _sc_aggregate(nodes, adj, features):
  B = nodes.shape[0]
  DEG = adj.shape[1]
  D = features.shape[1]
  assert B % NW == 0
  b_per_w = B // NW
  qrows = b_per_w // NQ

  mesh = plsc.VectorSubcoreMesh(core_axis_name="c", subcore_axis_name="s",
                                num_cores=NC, num_subcores=NS)

  @functools.partial(
      pl.kernel,
      mesh=mesh,
      compiler_params=pltpu.CompilerParams(use_tc_tiling_on_sc=False,
                                          needs_layout_passes=False),
      out_type=jax.ShapeDtypeStruct((B, D), jnp.float32),
      scratch_types=[
          pltpu.VMEM((b_per_w,), jnp.int32),
          pltpu.VMEM((b_per_w, DEG), jnp.int32),
          pltpu.VMEM((DEG, b_per_w), jnp.int32),
          pltpu.VMEM((NQ, qrows, D), jnp.float32),
          pltpu.SemaphoreType.DMA,
          pltpu.SemaphoreType.DMA((NQ,)),
      ],
  )
  def agg_kernel(nodes_hbm, adj_hbm, feat_hbm, out_hbm,
                 nodes_v, adjrows_v, adjt_v, acc_v, sem0, qsems):
    wid = lax.axis_index("s") * NC + lax.axis_index("c")
    base = wid * b_per_w

    pltpu.sync_copy(nodes_hbm.at[pl.ds(base, b_per_w)], nodes_v)
    pltpu.async_copy(adj_hbm.at[nodes_v], adjrows_v, sem0).wait()

    lane = lax.iota(jnp.int32, LANES)

    def tr_body(i, _):
      rows = i * LANES + lane
      for r in range(DEG):
        cols = jnp.full((LANES,), r, jnp.int32)
        vals = plsc.load_gather(adjrows_v, [rows, cols])
        adjt_v[r, pl.ds(i * LANES, LANES)] = vals
      return 0

    lax.fori_loop(0, b_per_w // LANES, tr_body, 0)

    def gadd(r, q, add):
      pltpu.async_copy(
          feat_hbm.at[adjt_v.at[r, pl.ds(q * qrows, qrows)]], acc_v.at[q],
          qsems.at[q], add=add)

    for q in range(NQ):
      gadd(0, q, False)

    def r_body(r, _):
      for q in range(NQ):
        pltpu.make_async_copy(
            feat_hbm.at[adjt_v.at[0, pl.ds(q * qrows, qrows)]], acc_v.at[q],
            qsems.at[q]).wait()

        @pl.when(r < DEG)
        def _():
          gadd(r, q, True)
      return 0

    lax.fori_loop(1, DEG + 1, r_body, 0)

    for q in range(NQ):
      pltpu.sync_copy(acc_v.at[q], out_hbm.at[pl.ds(base + q * qrows, qrows)])

  return agg_kernel(nodes, adj, features)


def _tc_tail(agg, W, scale):
  B, D = agg.shape
  BLK = 2048
  grid = B // BLK

  def body(a_ref, w_ref, o_ref):
    a = a_ref[...] * scale
    h = jnp.dot(a, w_ref[...], preferred_element_type=jnp.float32)
    h = jnp.maximum(h, 0.0)
    norm = jnp.sqrt(jnp.sum(h * h, axis=1, keepdims=True))
    o_ref[...] = h / jnp.maximum(norm, 1e-12)

  return pl.pallas_call(
      body,
      grid=(grid,),
      in_specs=[
          pl.BlockSpec((BLK, D), lambda i: (i, 0)),
          pl.BlockSpec((D, D), lambda i: (0, 0)),
      ],
      out_specs=pl.BlockSpec((BLK, D), lambda i: (i, 0)),
      out_shape=jax.ShapeDtypeStruct((B, D), jnp.float32),
  )(agg, W)


@jax.jit
def kernel(nodes, features, adj, W):
  nodes = nodes.astype(jnp.int32)
  aggsum = _sc_aggregate(nodes, adj, features)
  return _tc_tail(aggsum, W, 1.0 / adj.shape[1])


# speedup vs baseline: 1.9044x; 1.0463x over previous
import functools

import jax
import jax.numpy as jnp
from jax import lax
from jax.experimental import pallas as pl
from jax.experimental.pallas import tpu as pltpu
from jax.experimental.pallas import tpu_sc as plsc

NC = 2
NS = 16
NW = NC * NS
LANES = 16
NQ = 8


def _sc_aggregate(nodes, adj, features):
  B = nodes.shape[0]
  DEG = adj.shape[1]
  D = features.shape[1]
  assert B % NW == 0
  b_per_w = B // NW
  qrows = b_per_w // NQ

  mesh = plsc.VectorSubcoreMesh(core_axis_name="c", subcore_axis_name="s",
                                num_cores=NC, num_subcores=NS)

  @functools.partial(
      pl.kernel,
      mesh=mesh,
      compiler_params=pltpu.CompilerParams(use_tc_tiling_on_sc=False,
                                          needs_layout_passes=False),
      out_type=jax.ShapeDtypeStruct((B, D), jnp.float32),
      scratch_types=[
          pltpu.VMEM((b_per_w,), jnp.int32),
          pltpu.VMEM((b_per_w, DEG), jnp.int32),
          pltpu.VMEM((DEG, b_per_w), jnp.int32),
          pltpu.VMEM((NQ, qrows, D), jnp.float32),
          pltpu.SemaphoreType.DMA,
          pltpu.SemaphoreType.DMA((NQ,)),
      ],
  )
  def agg_kernel(nodes_hbm, adj_hbm, feat_hbm, out_hbm,
                 nodes_v, adjrows_v, adjt_v, acc_v, sem0, qsems):
    wid = lax.axis_index("s") * NC + lax.axis_index("c")
    base = wid * b_per_w

    pltpu.sync_copy(nodes_hbm.at[pl.ds(base, b_per_w)], nodes_v)
    pltpu.async_copy(adj_hbm.at[nodes_v], adjrows_v, sem0).wait()

    lane = lax.iota(jnp.int32, LANES)

    def tr_body(i, _):
      rows = i * LANES + lane
      for r in range(DEG):
        cols = jnp.full((LANES,), r, jnp.int32)
        vals = plsc.load_gather(adjrows_v, [rows, cols])
        adjt_v[r, pl.ds(i * LANES, LANES)] = vals
      return 0

    lax.fori_loop(0, b_per_w // LANES, tr_body, 0)

    def gadd(r, q, add):
      pltpu.async_copy(
          feat_hbm.at[adjt_v.at[r, pl.ds(q * qrows, qrows)]], acc_v.at[q],
          qsems.at[q], add=add)

    for q in range(NQ):
      gadd(0, q, False)

    def r_body(r, _):
      for q in range(NQ):
        pltpu.make_async_copy(
            feat_hbm.at[adjt_v.at[0, pl.ds(q * qrows, qrows)]], acc_v.at[q],
            qsems.at[q]).wait()

        @pl.when(r < DEG)
        def _():
          gadd(r, q, True)
      return 0

    lax.fori_loop(1, DEG + 1, r_body, 0)

    for q in range(NQ):
      pltpu.sync_copy(acc_v.at[q], out_hbm.at[pl.ds(base + q * qrows, qrows)])

  return agg_kernel(nodes, adj, features)


def _tc_tail(agg, W, scale):
  B, D = agg.shape
  BLK = 2048
  grid = B // BLK

  def body(a_ref, w_ref, o_ref):
    a = a_ref[...] * scale
    h = jnp.dot(a, w_ref[...], preferred_element_type=jnp.float32)
    h = jnp.maximum(h, 0.0)
    norm = jnp.sqrt(jnp.sum(h * h, axis=1, keepdims=True))
    o_ref[...] = h / jnp.maximum(norm, 1e-12)

  return pl.pallas_call(
      body,
      grid=(grid,),
      in_specs=[
          pl.BlockSpec((BLK, D), lambda i: (i, 0)),
          pl.BlockSpec((D, D), lambda i: (0, 0)),
      ],
      out_specs=pl.BlockSpec((BLK, D), lambda i: (i, 0)),
      out_shape=jax.ShapeDtypeStruct((B, D), jnp.float32),
  )(agg, W)


@jax.jit
def kernel(nodes, features, adj, W):
  nodes = nodes.astype(jnp.int32)
  aggsum = _sc_aggregate(nodes, adj, features)
  return _tc_tail(aggsum, W, 1.0 / adj.shape[1])
